# Initial kernel scaffold; baseline (speedup 1.0000x reference)
import jax
import jax.numpy as jnp
from jax.experimental import pallas as pl


def kernel(local_env, pair_indices, bond_features, W_az, b_az, W_cat, b_cat):
    raise NotImplementedError("write your pallas kernel here")


# speedup vs baseline: 163.6974x; 163.6974x over previous
import functools

import jax
import jax.numpy as jnp
from jax import lax
from jax.experimental import pallas as pl
from jax.experimental.pallas import tpu as pltpu
from jax.experimental.pallas import tpu_sc as plsc

E = 10000
N_NODES = 2500
BD = 128
EP = 10240
NW = 32
EPW = EP // NW
CH = 64
NCH = EPW // CH
NR = 2560
ZR = NR // 16
OFFP = 2504

_ATAN_C = (0.9999994160035325, -0.3333022235532037, 0.19951110891900356,
           -0.13933229393279548, 0.09709350737147433, -0.05688089274197976,
           0.02256682612663299, -0.004257409078051173)
_PI = 3.14159265358979
_PI_2 = 1.570796326794897


def _sqrt16(x):
    i = plsc.bitcast(x, jnp.int32)
    y = plsc.bitcast(jnp.int32(0x1FBD1DF5) + lax.shift_right_arithmetic(i, 1),
                     jnp.float32)
    half = jnp.float32(0.5)
    for _ in range(3):
        y = half * (y + x / jnp.where(y == 0.0, jnp.float32(1.0), y))
    return jnp.where(x <= 0.0, jnp.float32(0.0), y)


def _atan2_pos(y, x):
    ax = jnp.abs(x)
    mx = jnp.maximum(ax, y)
    mn = jnp.minimum(ax, y)
    t = mn / jnp.where(mx == 0.0, jnp.float32(1.0), mx)
    u = t * t
    p = jnp.float32(_ATAN_C[7])
    for c in _ATAN_C[6::-1]:
        p = p * u + jnp.float32(c)
    r = t * p
    r = jnp.where(y > ax, jnp.float32(_PI_2) - r, r)
    r = jnp.where(x < 0.0, jnp.float32(_PI) - r, r)
    return jnp.where(mx == 0.0, jnp.float32(0.0), r)


def _tc1_body(bond_ref, wtop_ref, m8_ref, wbot_ref, bcat_ref, bf2_ref, aux_ref):
    bf2_ref[...] = jnp.dot(bond_ref[...], wtop_ref[...],
                           preferred_element_type=jnp.float32)

    @pl.when(pl.program_id(0) == 0)
    def _():
        mm = jnp.dot(m8_ref[...], wbot_ref[...],
                     preferred_element_type=jnp.float32)
        rowid = lax.broadcasted_iota(jnp.int32, (8, 1), 0)
        aux_ref[...] = mm + jnp.where(rowid == 3, jnp.float32(1.0),
                                      jnp.float32(0.0)) * bcat_ref[...]


def _tc1(bond_p, wtop, m8, wbot, bcat_row):
    blk = 512
    return pl.pallas_call(
        _tc1_body,
        grid=(EP // blk,),
        in_specs=[
            pl.BlockSpec((blk, BD), lambda i: (i, 0)),
            pl.BlockSpec((BD, BD), lambda i: (0, 0)),
            pl.BlockSpec((8, 64), lambda i: (0, 0)),
            pl.BlockSpec((64, BD), lambda i: (0, 0)),
            pl.BlockSpec((1, BD), lambda i: (0, 0)),
        ],
        out_specs=[
            pl.BlockSpec((blk, BD), lambda i: (i, 0)),
            pl.BlockSpec((8, BD), lambda i: (0, 0)),
        ],
        out_shape=[
            jax.ShapeDtypeStruct((EP, BD), jnp.float32),
            jax.ShapeDtypeStruct((8, BD), jnp.float32),
        ],
    )(bond_p, wtop, m8, wbot, bcat_row)


def _sc2_body(bf2, recvs, sends, px_h, py_h, pz_h, vx_h, vy_h, vz_h,
              ex_h, ey_h, ez_h, perm, offs,
              partials, pout0, pout1, pout2,
              sh_acc, v_bf2, v_zero, v_idx, v_ex, v_ey, v_ez, v_perm, v_off,
              v_le0, v_le1, v_le2, v_le3, v_le4, v_le5, v_send,
              v_p0, v_p1, v_p2):
    cid = lax.axis_index("c")
    sid = lax.axis_index("s")
    wid = cid * 16 + sid
    base = wid * EPW

    zvec = jnp.zeros((16,), jnp.float32)
    for r in range(8):
        for k in range(BD // 16):
            v_zero[r, pl.ds(k * 16, 16)] = zvec

    def _zero_fill(t, _):
        pltpu.sync_copy(v_zero, sh_acc.at[pl.ds(sid * ZR + t * 8, 8)])
        return 0

    lax.fori_loop(0, ZR // 8, _zero_fill, 0)
    plsc.subcore_barrier()

    pltpu.sync_copy(bf2.at[pl.ds(base, EPW)], v_bf2)
    for j in range(NCH):
        pltpu.sync_copy(recvs.at[pl.ds(base + j * CH, CH)], v_idx.at[j])
    for j in range(NCH):
        pltpu.sync_copy(v_bf2.at[pl.ds(j * CH, CH)], sh_acc.at[v_idx.at[j]],
                        add=True)
    plsc.subcore_barrier()

    pltpu.sync_copy(sh_acc.at[pl.ds(sid * ZR, ZR)],
                    partials.at[cid, pl.ds(sid * ZR, ZR)])

    pltpu.sync_copy(ex_h, v_ex)
    pltpu.sync_copy(ey_h, v_ey)
    pltpu.sync_copy(ez_h, v_ez)
    pltpu.sync_copy(perm, v_perm)
    pltpu.sync_copy(offs, v_off)
    lsl = pl.ds(base, EPW)
    for src, dst in ((px_h, v_le0), (py_h, v_le1), (pz_h, v_le2),
                     (vx_h, v_le3), (vy_h, v_le4), (vz_h, v_le5)):
        pltpu.sync_copy(src.at[lsl], dst)
    pltpu.sync_copy(sends.at[lsl], v_send)

    for g in range(EPW // 16):
        sl = pl.ds(g * 16, 16)
        n = v_send[sl]
        lo = plsc.load_gather(v_off, [n])
        hi = plsc.load_gather(v_off, [n + 1])
        px = v_le0[sl]
        py = v_le1[sl]
        pz = v_le2[sl]
        vx = v_le3[sl]
        vy = v_le4[sl]
        vz = v_le5[sl]
        ln = hi - lo
        maxlen = jnp.max(ln)

        def _cond(carry):
            return carry[0] < maxlen

        def _step(carry):
            t, accp, acct = carry
            j = lo + t
            msk = j < hi
            jc = jnp.where(msk, j, 0)
            e = plsc.load_gather(v_perm, [jc])
            ex = plsc.load_gather(v_ex, [e])
            ey = plsc.load_gather(v_ey, [e])
            ez = plsc.load_gather(v_ez, [e])
            cx = py * ez - pz * ey
            cy = pz * ex - px * ez
            cz = px * ey - py * ex
            c = _sqrt16(cx * cx + cy * cy + cz * cz)
            d1 = px * ex + py * ey + pz * ez
            d2 = vx * ex + vy * ey + vz * ez
            theta = _atan2_pos(c, d1)
            phi = _atan2_pos(jnp.abs(d1) * c, d1 * d2)
            zero = jnp.float32(0.0)
            accp = accp + jnp.where(msk, phi, zero)
            acct = acct + jnp.where(msk, theta, zero)
            return t + 1, accp, acct

        z16 = jnp.zeros((16,), jnp.float32)
        _, accp, acct = lax.while_loop(_cond, _step, (jnp.int32(0), z16, z16))
        v_p0[sl] = accp
        v_p1[sl] = acct
        v_p2[sl] = ln.astype(jnp.float32)

    pltpu.sync_copy(v_p0, pout0.at[pl.ds(base, EPW)])
    pltpu.sync_copy(v_p1, pout1.at[pl.ds(base, EPW)])
    pltpu.sync_copy(v_p2, pout2.at[pl.ds(base, EPW)])


def _sc2(bf2, recvs_p, sends_p, le_cols, edge_cols, perm, offs):
    mesh = plsc.VectorSubcoreMesh(core_axis_name="c", subcore_axis_name="s")
    return pl.kernel(
        _sc2_body,
        out_type=[
            jax.ShapeDtypeStruct((2, NR, BD), jnp.float32),
            jax.ShapeDtypeStruct((EP,), jnp.float32),
            jax.ShapeDtypeStruct((EP,), jnp.float32),
            jax.ShapeDtypeStruct((EP,), jnp.float32),
        ],
        mesh=mesh,
        compiler_params=pltpu.CompilerParams(needs_layout_passes=False),
        scratch_types=[
            pltpu.VMEM_SHARED((NR, BD), jnp.float32),
            pltpu.VMEM((EPW, BD), jnp.float32),
            pltpu.VMEM((8, BD), jnp.float32),
            pltpu.VMEM((NCH, CH), jnp.int32),
            pltpu.VMEM((E,), jnp.float32),
            pltpu.VMEM((E,), jnp.float32),
            pltpu.VMEM((E,), jnp.float32),
            pltpu.VMEM((E,), jnp.int32),
            pltpu.VMEM((OFFP,), jnp.int32),
            pltpu.VMEM((EPW,), jnp.float32),
            pltpu.VMEM((EPW,), jnp.float32),
            pltpu.VMEM((EPW,), jnp.float32),
            pltpu.VMEM((EPW,), jnp.float32),
            pltpu.VMEM((EPW,), jnp.float32),
            pltpu.VMEM((EPW,), jnp.float32),
            pltpu.VMEM((EPW,), jnp.int32),
            pltpu.VMEM((EPW,), jnp.float32),
            pltpu.VMEM((EPW,), jnp.float32),
            pltpu.VMEM((EPW,), jnp.float32),
        ],
    )(bf2, recvs_p, sends_p, *le_cols, *edge_cols, perm, offs)


def _sc3_body(partials, sends, p0_h, p1_h, p2_h, aux, outp,
              v_r0, v_r1, v_idx, v_phi, v_th, v_cnt, v_aux, sem):
    cid = lax.axis_index("c")
    sid = lax.axis_index("s")
    wid = cid * 16 + sid
    base = wid * EPW

    for j in range(NCH):
        pltpu.sync_copy(sends.at[pl.ds(base + j * CH, CH)], v_idx.at[j])
    pltpu.sync_copy(p0_h.at[pl.ds(base, EPW)], v_phi.at[pl.ds(0, EPW)])
    pltpu.sync_copy(p1_h.at[pl.ds(base, EPW)], v_th.at[pl.ds(0, EPW)])
    pltpu.sync_copy(p2_h.at[pl.ds(base, EPW)], v_cnt.at[pl.ds(0, EPW)])
    pltpu.sync_copy(aux, v_aux)
    for j in range(NCH):
        pltpu.async_copy(partials.at[0].at[v_idx.at[j]],
                         v_r0.at[pl.ds(j * CH, CH)], sem).wait()
        pltpu.async_copy(partials.at[1].at[v_idx.at[j]],
                         v_r1.at[pl.ds(j * CH, CH)], sem).wait()

    a0 = [v_aux[0, pl.ds(k * 16, 16)] for k in range(BD // 16)]
    a1 = [v_aux[1, pl.ds(k * 16, 16)] for k in range(BD // 16)]
    a2 = [v_aux[2, pl.ds(k * 16, 16)] for k in range(BD // 16)]
    a3 = [v_aux[3, pl.ds(k * 16, 16)] for k in range(BD // 16)]

    def _row(r, _):
        sp = v_phi[pl.ds(r, 16)][0]
        st = v_th[pl.ds(r, 16)][0]
        sc = v_cnt[pl.ds(r, 16)][0]
        for k in range(BD // 16):
            sl = pl.ds(k * 16, 16)
            acc = v_r0[r, sl] + v_r1[r, sl]
            acc = acc + sp * a0[k] + st * a1[k] + sc * a2[k] + a3[k]
            v_r0[r, sl] = acc
        return 0

    lax.fori_loop(0, EPW, _row, 0)
    pltpu.sync_copy(v_r0, outp.at[pl.ds(base, EPW)])


def _sc3(partials, sends_p, p0, p1, p2, aux):
    mesh = plsc.VectorSubcoreMesh(core_axis_name="c", subcore_axis_name="s")
    return pl.kernel(
        _sc3_body,
        out_type=jax.ShapeDtypeStruct((EP, BD), jnp.float32),
        mesh=mesh,
        scratch_types=[
            pltpu.VMEM((EPW, BD), jnp.float32),
            pltpu.VMEM((EPW, BD), jnp.float32),
            pltpu.VMEM((NCH, CH), jnp.int32),
            pltpu.VMEM((EPW + 16,), jnp.float32),
            pltpu.VMEM((EPW + 16,), jnp.float32),
            pltpu.VMEM((EPW + 16,), jnp.float32),
            pltpu.VMEM((8, BD), jnp.float32),
            pltpu.SemaphoreType.DMA,
        ],
    )(partials, sends_p, p0, p1, p2, aux)


def kernel(local_env, pair_indices, bond_features, W_az, b_az, W_cat, b_cat):
    sends = pair_indices[:, 0].astype(jnp.int32)
    recvs = pair_indices[:, 1].astype(jnp.int32)

    perm = jnp.argsort(recvs).astype(jnp.int32)
    sorted_r = recvs[perm]
    offs = jnp.searchsorted(sorted_r, jnp.arange(N_NODES + 1)).astype(jnp.int32)
    offs = jnp.pad(offs, (0, OFFP - (N_NODES + 1)))

    pad_e = EP - E
    bond_p = jnp.pad(bond_features, ((0, pad_e), (0, 0)))
    sends_p = jnp.pad(sends, (0, pad_e))
    recvs_p = jnp.pad(recvs, (0, pad_e))
    le_cols = tuple(jnp.pad(local_env[:, k], (0, pad_e)) for k in range(6))
    edge_cols = tuple(local_env[:, k] for k in range(3))

    wtop = W_cat[:BD]
    wbot = W_cat[BD:]
    m8 = jnp.zeros((8, 64), jnp.float32).at[0:2].set(W_az).at[2].set(b_az)
    bcat_row = b_cat.reshape(1, BD)

    bf2, aux = _tc1(bond_p, wtop, m8, wbot, bcat_row)
    partials, p0, p1, p2 = _sc2(bf2, recvs_p, sends_p, le_cols, edge_cols,
                                perm, offs)
    outp = _sc3(partials, sends_p, p0, p1, p2, aux)
    return outp[:E]


# speedup vs baseline: 383.4343x; 2.3423x over previous
import functools

import jax
import jax.numpy as jnp
from jax import lax
from jax.experimental import pallas as pl
from jax.experimental.pallas import tpu as pltpu
from jax.experimental.pallas import tpu_sc as plsc

E = 10000
N_NODES = 2500
BD = 128
EP = 10240
NW = 32
EPW = EP // NW
CH = 64
NCH = EPW // CH
NR = 2560
ZR = NR // 16
OFFP = 2504

_ATAN_C = (0.9999994160035325, -0.3333022235532037, 0.19951110891900356,
           -0.13933229393279548, 0.09709350737147433, -0.05688089274197976,
           0.02256682612663299, -0.004257409078051173)
_PI = 3.14159265358979
_PI_2 = 1.570796326794897


def _sqrt16(x):
    i = plsc.bitcast(x, jnp.int32)
    y = plsc.bitcast(jnp.int32(0x1FBD1DF5) + lax.shift_right_arithmetic(i, 1),
                     jnp.float32)
    half = jnp.float32(0.5)
    for _ in range(3):
        y = half * (y + x / jnp.where(y == 0.0, jnp.float32(1.0), y))
    return jnp.where(x <= 0.0, jnp.float32(0.0), y)


def _atan2_pos(y, x):
    ax = jnp.abs(x)
    mx = jnp.maximum(ax, y)
    mn = jnp.minimum(ax, y)
    t = mn / jnp.where(mx == 0.0, jnp.float32(1.0), mx)
    u = t * t
    p = jnp.float32(_ATAN_C[7])
    for c in _ATAN_C[6::-1]:
        p = p * u + jnp.float32(c)
    r = t * p
    r = jnp.where(y > ax, jnp.float32(_PI_2) - r, r)
    r = jnp.where(x < 0.0, jnp.float32(_PI) - r, r)
    return jnp.where(mx == 0.0, jnp.float32(0.0), r)


def _tc1_body(bond_ref, wtop_ref, m8_ref, wbot_ref, bcat_ref, bf2_ref, aux_ref):
    bf2_ref[...] = jnp.dot(bond_ref[...], wtop_ref[...],
                           preferred_element_type=jnp.float32)

    @pl.when(pl.program_id(0) == 0)
    def _():
        mm = jnp.dot(m8_ref[...], wbot_ref[...],
                     preferred_element_type=jnp.float32)
        rowid = lax.broadcasted_iota(jnp.int32, (8, 1), 0)
        aux_ref[...] = mm + jnp.where(rowid == 3, jnp.float32(1.0),
                                      jnp.float32(0.0)) * bcat_ref[...]


def _tc1(bond_p, wtop, m8, wbot, bcat_row):
    blk = 512
    return pl.pallas_call(
        _tc1_body,
        grid=(EP // blk,),
        in_specs=[
            pl.BlockSpec((blk, BD), lambda i: (i, 0)),
            pl.BlockSpec((BD, BD), lambda i: (0, 0)),
            pl.BlockSpec((8, 64), lambda i: (0, 0)),
            pl.BlockSpec((64, BD), lambda i: (0, 0)),
            pl.BlockSpec((1, BD), lambda i: (0, 0)),
        ],
        out_specs=[
            pl.BlockSpec((blk, BD), lambda i: (i, 0)),
            pl.BlockSpec((8, BD), lambda i: (0, 0)),
        ],
        out_shape=[
            jax.ShapeDtypeStruct((EP, BD), jnp.float32),
            jax.ShapeDtypeStruct((8, BD), jnp.float32),
        ],
    )(bond_p, wtop, m8, wbot, bcat_row)


NE6 = 640
HP = 2560
HB = 640
PCH = 2048


def _sc2_body(bf2, recvs, sends, px_h, py_h, pz_h, vx_h, vy_h, vz_h,
              ex_h, ey_h, ez_h,
              partials, pout0, pout1, pout2,
              sh_acc, sh_hist, sh_pos, sh_val,
              v_bf2, v_zero, v_idx, v_ex, v_ey, v_ez, v_perm,
              v_off, v_start, v_hist, v_allh, v_recv6, v_key6, v_val6,
              v_rank6, v_pos6, v_tmp16, v_pch, v_vch,
              v_le0, v_le1, v_le2, v_le3, v_le4, v_le5, v_send,
              v_p0, v_p1, v_p2, sem):
    cid = lax.axis_index("c")
    sid = lax.axis_index("s")
    wid = cid * 16 + sid
    base = wid * EPW
    base6 = sid * NE6
    iota = lax.broadcasted_iota(jnp.int32, (16,), 0)
    z16 = jnp.zeros((16,), jnp.float32)
    z16i = jnp.zeros((16,), jnp.int32)

    for r in range(16):
        for k in range(BD // 16):
            v_zero[r, pl.ds(k * 16, 16)] = z16
    for t in range(ZR // 16):
        pltpu.sync_copy(v_zero, sh_acc.at[pl.ds(sid * ZR + t * 16, 16)])

    pltpu.sync_copy(recvs.at[pl.ds(base6, NE6)], v_recv6)
    def _hzero(i, _):
        v_hist[pl.ds(i * 16, 16)] = z16i
        return 0

    lax.fori_loop(0, HP // 16, _hzero, 0)

    def _count(i, _):
        sl6 = pl.ds(i * 16, 16)
        k16 = v_recv6[sl6]
        e16 = base6 + i * 16 + iota
        ks, vs = plsc.sort_key_val(k16, e16)
        v_tmp16[...] = ks
        prev = plsc.load_gather(v_tmp16, [jnp.maximum(iota - 1, 0)])
        nxt = plsc.load_gather(v_tmp16, [jnp.minimum(iota + 1, 15)])
        isnew = (iota == 0) | (ks != prev)
        rk = iota - plsc.cummax(jnp.where(isnew, iota, 0))
        last = (iota == 15) | (ks != nxt)
        plsc.addupdate_scatter(v_hist, [ks], rk + 1, mask=last)
        v_key6[sl6] = ks
        v_val6[sl6] = vs
        v_rank6[sl6] = rk
        return 0

    lax.fori_loop(0, NE6 // 16, _count, 0)
    pltpu.sync_copy(v_hist, sh_hist.at[pl.ds(sid * HP, HP)])
    plsc.subcore_barrier()

    for j in range(NCH):
        pltpu.sync_copy(recvs.at[pl.ds(base + j * CH, CH)], v_idx.at[j])
    for j in range(NCH):
        pltpu.sync_copy(bf2.at[pl.ds(base + j * CH, CH)], v_bf2)
        pltpu.sync_copy(v_bf2, sh_acc.at[v_idx.at[j]], add=True)

    carry = jnp.int32(0)
    for b in range(HP // HB):
        descs = [pltpu.async_copy(sh_hist.at[pl.ds(sp * HP + b * HB, HB)],
                                  v_allh.at[pl.ds(sp * HB, HB)], sem)
                 for sp in range(16)]
        for d in descs:
            d.wait()

        def _prefix(i, c):
            tot = z16i
            mysum = z16i
            for sp in range(16):
                h = v_allh[pl.ds(sp * HB + i * 16, 16)]
                tot = tot + h
                mysum = mysum + jnp.where(sp < sid, h, z16i)
            cum = plsc.cumsum(tot)
            excl = c + cum - tot
            v_off[pl.ds(b * HB + i * 16, 16)] = excl
            v_start[pl.ds(b * HB + i * 16, 16)] = excl + mysum
            return c + cum[15]

        carry = lax.fori_loop(0, HB // 16, _prefix, carry)

    def _place(i, _):
        sl6 = pl.ds(i * 16, 16)
        ks = v_key6[sl6]
        rk = v_rank6[sl6]
        st = plsc.load_gather(v_start, [ks])
        v_pos6[sl6] = st + rk
        v_tmp16[...] = ks
        nxt = plsc.load_gather(v_tmp16, [jnp.minimum(iota + 1, 15)])
        last = (iota == 15) | (ks != nxt)
        plsc.addupdate_scatter(v_start, [ks], rk + 1, mask=last)
        return 0

    lax.fori_loop(0, NE6 // 16, _place, 0)
    pltpu.sync_copy(v_pos6, sh_pos.at[pl.ds(base6, NE6)])
    pltpu.sync_copy(v_val6, sh_val.at[pl.ds(base6, NE6)])
    plsc.subcore_barrier()

    pltpu.sync_copy(sh_acc.at[pl.ds(sid * ZR, ZR)],
                    partials.at[cid, pl.ds(sid * ZR, ZR)])

    for j in range(EP // PCH):
        pltpu.sync_copy(sh_pos.at[pl.ds(j * PCH, PCH)], v_pch)
        pltpu.sync_copy(sh_val.at[pl.ds(j * PCH, PCH)], v_vch)

        def _scat(i, _):
            slc = pl.ds(i * 16, 16)
            plsc.store_scatter(v_perm, [v_pch[slc]], v_vch[slc])
            return 0

        lax.fori_loop(0, PCH // 16, _scat, 0)

    pltpu.sync_copy(ex_h, v_ex)
    pltpu.sync_copy(ey_h, v_ey)
    pltpu.sync_copy(ez_h, v_ez)
    lsl = pl.ds(base, EPW)
    for src, dst in ((px_h, v_le0), (py_h, v_le1), (pz_h, v_le2),
                     (vx_h, v_le3), (vy_h, v_le4), (vz_h, v_le5)):
        pltpu.sync_copy(src.at[lsl], dst)
    pltpu.sync_copy(sends.at[lsl], v_send)

    def _group(g, _):
        sl = pl.ds(g * 16, 16)
        n = v_send[sl]
        lo = plsc.load_gather(v_off, [n])
        hi = plsc.load_gather(v_off, [n + 1])
        px = v_le0[sl]
        py = v_le1[sl]
        pz = v_le2[sl]
        vx = v_le3[sl]
        vy = v_le4[sl]
        vz = v_le5[sl]
        ln = hi - lo
        maxlen = jnp.max(ln)

        def _cond(carry):
            return carry[0] < maxlen

        def _step(carry):
            t, accp, acct = carry
            j = lo + t
            msk = j < hi
            jc = jnp.where(msk, j, 0)
            e = plsc.load_gather(v_perm, [jc])
            ex = plsc.load_gather(v_ex, [e])
            ey = plsc.load_gather(v_ey, [e])
            ez = plsc.load_gather(v_ez, [e])
            cx = py * ez - pz * ey
            cy = pz * ex - px * ez
            cz = px * ey - py * ex
            c = _sqrt16(cx * cx + cy * cy + cz * cz)
            d1 = px * ex + py * ey + pz * ez
            d2 = vx * ex + vy * ey + vz * ez
            theta = _atan2_pos(c, d1)
            phi = _atan2_pos(jnp.abs(d1) * c, d1 * d2)
            zero = jnp.float32(0.0)
            accp = accp + jnp.where(msk, phi, zero)
            acct = acct + jnp.where(msk, theta, zero)
            return t + 1, accp, acct

        _, accp, acct = lax.while_loop(_cond, _step, (jnp.int32(0), z16, z16))
        v_p0[sl] = accp
        v_p1[sl] = acct
        v_p2[sl] = ln.astype(jnp.float32)
        return 0

    lax.fori_loop(0, EPW // 16, _group, 0)

    pltpu.sync_copy(v_p0, pout0.at[pl.ds(base, EPW)])
    pltpu.sync_copy(v_p1, pout1.at[pl.ds(base, EPW)])
    pltpu.sync_copy(v_p2, pout2.at[pl.ds(base, EPW)])


def _sc2(bf2, recvs_p, sends_p, le_cols, edge_cols):
    mesh = plsc.VectorSubcoreMesh(core_axis_name="c", subcore_axis_name="s")
    return pl.kernel(
        _sc2_body,
        out_type=[
            jax.ShapeDtypeStruct((2, NR, BD), jnp.float32),
            jax.ShapeDtypeStruct((EP,), jnp.float32),
            jax.ShapeDtypeStruct((EP,), jnp.float32),
            jax.ShapeDtypeStruct((EP,), jnp.float32),
        ],
        mesh=mesh,
        compiler_params=pltpu.CompilerParams(needs_layout_passes=False),
        scratch_types=[
            pltpu.VMEM_SHARED((NR, BD), jnp.float32),
            pltpu.VMEM_SHARED((16 * HP,), jnp.int32),
            pltpu.VMEM_SHARED((EP,), jnp.int32),
            pltpu.VMEM_SHARED((EP,), jnp.int32),
            pltpu.VMEM((CH, BD), jnp.float32),
            pltpu.VMEM((16, BD), jnp.float32),
            pltpu.VMEM((NCH, CH), jnp.int32),
            pltpu.VMEM((E,), jnp.float32),
            pltpu.VMEM((E,), jnp.float32),
            pltpu.VMEM((E,), jnp.float32),
            pltpu.VMEM((EP,), jnp.int32),
            pltpu.VMEM((HP,), jnp.int32),
            pltpu.VMEM((HP,), jnp.int32),
            pltpu.VMEM((HP,), jnp.int32),
            pltpu.VMEM((16 * HB,), jnp.int32),
            pltpu.VMEM((NE6,), jnp.int32),
            pltpu.VMEM((NE6,), jnp.int32),
            pltpu.VMEM((NE6,), jnp.int32),
            pltpu.VMEM((NE6,), jnp.int32),
            pltpu.VMEM((NE6,), jnp.int32),
            pltpu.VMEM((16,), jnp.int32),
            pltpu.VMEM((PCH,), jnp.int32),
            pltpu.VMEM((PCH,), jnp.int32),
            pltpu.VMEM((EPW,), jnp.float32),
            pltpu.VMEM((EPW,), jnp.float32),
            pltpu.VMEM((EPW,), jnp.float32),
            pltpu.VMEM((EPW,), jnp.float32),
            pltpu.VMEM((EPW,), jnp.float32),
            pltpu.VMEM((EPW,), jnp.float32),
            pltpu.VMEM((EPW,), jnp.int32),
            pltpu.VMEM((EPW,), jnp.float32),
            pltpu.VMEM((EPW,), jnp.float32),
            pltpu.VMEM((EPW,), jnp.float32),
            pltpu.SemaphoreType.DMA,
        ],
    )(bf2, recvs_p, sends_p, *le_cols, *edge_cols)


def _sc3_body(partials, sends, p0_h, p1_h, p2_h, aux, outp,
              v_r0, v_r1, v_idx, v_phi, v_th, v_cnt, v_aux, sem):
    cid = lax.axis_index("c")
    sid = lax.axis_index("s")
    wid = cid * 16 + sid
    base = wid * EPW

    for j in range(NCH):
        pltpu.sync_copy(sends.at[pl.ds(base + j * CH, CH)], v_idx.at[j])
    pltpu.sync_copy(p0_h.at[pl.ds(base, EPW)], v_phi.at[pl.ds(0, EPW)])
    pltpu.sync_copy(p1_h.at[pl.ds(base, EPW)], v_th.at[pl.ds(0, EPW)])
    pltpu.sync_copy(p2_h.at[pl.ds(base, EPW)], v_cnt.at[pl.ds(0, EPW)])
    pltpu.sync_copy(aux, v_aux)
    for j in range(NCH):
        pltpu.async_copy(partials.at[0].at[v_idx.at[j]],
                         v_r0.at[pl.ds(j * CH, CH)], sem).wait()
        pltpu.async_copy(partials.at[1].at[v_idx.at[j]],
                         v_r1.at[pl.ds(j * CH, CH)], sem).wait()

    a0 = [v_aux[0, pl.ds(k * 16, 16)] for k in range(BD // 16)]
    a1 = [v_aux[1, pl.ds(k * 16, 16)] for k in range(BD // 16)]
    a2 = [v_aux[2, pl.ds(k * 16, 16)] for k in range(BD // 16)]
    a3 = [v_aux[3, pl.ds(k * 16, 16)] for k in range(BD // 16)]

    def _row(r, _):
        sp = v_phi[pl.ds(r, 16)][0]
        st = v_th[pl.ds(r, 16)][0]
        sc = v_cnt[pl.ds(r, 16)][0]
        for k in range(BD // 16):
            sl = pl.ds(k * 16, 16)
            acc = v_r0[r, sl] + v_r1[r, sl]
            acc = acc + sp * a0[k] + st * a1[k] + sc * a2[k] + a3[k]
            v_r0[r, sl] = acc
        return 0

    lax.fori_loop(0, EPW, _row, 0)
    pltpu.sync_copy(v_r0, outp.at[pl.ds(base, EPW)])


def _sc3(partials, sends_p, p0, p1, p2, aux):
    mesh = plsc.VectorSubcoreMesh(core_axis_name="c", subcore_axis_name="s")
    return pl.kernel(
        _sc3_body,
        out_type=jax.ShapeDtypeStruct((EP, BD), jnp.float32),
        mesh=mesh,
        scratch_types=[
            pltpu.VMEM((EPW, BD), jnp.float32),
            pltpu.VMEM((EPW, BD), jnp.float32),
            pltpu.VMEM((NCH, CH), jnp.int32),
            pltpu.VMEM((EPW + 16,), jnp.float32),
            pltpu.VMEM((EPW + 16,), jnp.float32),
            pltpu.VMEM((EPW + 16,), jnp.float32),
            pltpu.VMEM((8, BD), jnp.float32),
            pltpu.SemaphoreType.DMA,
        ],
    )(partials, sends_p, p0, p1, p2, aux)


def kernel(local_env, pair_indices, bond_features, W_az, b_az, W_cat, b_cat):
    sends = pair_indices[:, 0].astype(jnp.int32)
    recvs = pair_indices[:, 1].astype(jnp.int32)

    pad_e = EP - E
    bond_p = jnp.pad(bond_features, ((0, pad_e), (0, 0)))
    sends_p = jnp.pad(sends, (0, pad_e))
    recvs_p = jnp.pad(recvs, (0, pad_e), constant_values=N_NODES)
    le_cols = tuple(jnp.pad(local_env[:, k], (0, pad_e)) for k in range(6))
    edge_cols = tuple(local_env[:, k] for k in range(3))

    wtop = W_cat[:BD]
    wbot = W_cat[BD:]
    m8 = jnp.zeros((8, 64), jnp.float32).at[0:2].set(W_az).at[2].set(b_az)
    bcat_row = b_cat.reshape(1, BD)

    bf2, aux = _tc1(bond_p, wtop, m8, wbot, bcat_row)
    partials, p0, p1, p2 = _sc2(bf2, recvs_p, sends_p, le_cols, edge_cols)
    outp = _sc3(partials, sends_p, p0, p1, p2, aux)
    return outp[:E]


# speedup vs baseline: 453.3291x; 1.1823x over previous
import functools

import jax
import jax.numpy as jnp
from jax import lax
from jax.experimental import pallas as pl
from jax.experimental.pallas import tpu as pltpu
from jax.experimental.pallas import tpu_sc as plsc

E = 10000
N_NODES = 2500
BD = 128
EP = 10240
NW = 32
EPW = EP // NW
CH = 64
NCH = EPW // CH
NR = 2560
ZR = NR // 16
OFFP = 2504

_ATAN_C = (0.9999994160035325, -0.3333022235532037, 0.19951110891900356,
           -0.13933229393279548, 0.09709350737147433, -0.05688089274197976,
           0.02256682612663299, -0.004257409078051173)
_PI = 3.14159265358979
_PI_2 = 1.570796326794897


def _sqrt16(x):
    i = plsc.bitcast(x, jnp.int32)
    y = plsc.bitcast(jnp.int32(0x1FBD1DF5) + lax.shift_right_arithmetic(i, 1),
                     jnp.float32)
    half = jnp.float32(0.5)
    for _ in range(3):
        y = half * (y + x / jnp.where(y == 0.0, jnp.float32(1.0), y))
    return jnp.where(x <= 0.0, jnp.float32(0.0), y)


def _atan2_pos(y, x):
    ax = jnp.abs(x)
    mx = jnp.maximum(ax, y)
    mn = jnp.minimum(ax, y)
    t = mn / jnp.where(mx == 0.0, jnp.float32(1.0), mx)
    u = t * t
    p = jnp.float32(_ATAN_C[7])
    for c in _ATAN_C[6::-1]:
        p = p * u + jnp.float32(c)
    r = t * p
    r = jnp.where(y > ax, jnp.float32(_PI_2) - r, r)
    r = jnp.where(x < 0.0, jnp.float32(_PI) - r, r)
    return jnp.where(mx == 0.0, jnp.float32(0.0), r)


def _tc1_body(bond_ref, wtop_ref, m8_ref, wbot_ref, bcat_ref, bf2_ref, aux_ref):
    bf2_ref[...] = jnp.dot(bond_ref[...], wtop_ref[...],
                           preferred_element_type=jnp.float32)

    @pl.when(pl.program_id(0) == 0)
    def _():
        mm = jnp.dot(m8_ref[...], wbot_ref[...],
                     preferred_element_type=jnp.float32)
        rowid = lax.broadcasted_iota(jnp.int32, (8, 1), 0)
        aux_ref[...] = mm + jnp.where(rowid == 3, jnp.float32(1.0),
                                      jnp.float32(0.0)) * bcat_ref[...]


def _tc1(bond_p, wtop, m8, wbot, bcat_row):
    blk = 512
    return pl.pallas_call(
        _tc1_body,
        grid=(EP // blk,),
        in_specs=[
            pl.BlockSpec((blk, BD), lambda i: (i, 0)),
            pl.BlockSpec((BD, BD), lambda i: (0, 0)),
            pl.BlockSpec((8, 64), lambda i: (0, 0)),
            pl.BlockSpec((64, BD), lambda i: (0, 0)),
            pl.BlockSpec((1, BD), lambda i: (0, 0)),
        ],
        out_specs=[
            pl.BlockSpec((blk, BD), lambda i: (i, 0)),
            pl.BlockSpec((8, BD), lambda i: (0, 0)),
        ],
        out_shape=[
            jax.ShapeDtypeStruct((EP, BD), jnp.float32),
            jax.ShapeDtypeStruct((8, BD), jnp.float32),
        ],
    )(bond_p, wtop, m8, wbot, bcat_row)


NE6 = 640
HP = 2560
HB = 640
PCH = 2048


def _sc2_body(bf2, recvs, sends, px_h, py_h, pz_h, vx_h, vy_h, vz_h,
              ex_h, ey_h, ez_h,
              partials, pout0, pout1, pout2,
              sh_acc, sh_hist, sh_pos, sh_val,
              v_bf2, v_zero, v_idx, v_ex, v_ey, v_ez, v_perm,
              v_off, v_start, v_hist, v_allh, v_recv6, v_key6, v_val6,
              v_rank6, v_pos6, v_tmp16, v_pch, v_vch,
              v_le0, v_le1, v_le2, v_le3, v_le4, v_le5, v_send,
              v_p0, v_p1, v_p2, sem):
    cid = lax.axis_index("c")
    sid = lax.axis_index("s")
    wid = cid * 16 + sid
    base = wid * EPW
    base6 = sid * NE6
    iota = lax.broadcasted_iota(jnp.int32, (16,), 0)
    z16 = jnp.zeros((16,), jnp.float32)
    z16i = jnp.zeros((16,), jnp.int32)

    pltpu.sync_copy(recvs.at[pl.ds(base6, NE6)], v_recv6)

    lsl = pl.ds(base, EPW)
    descs = [pltpu.async_copy(ex_h, v_ex, sem),
             pltpu.async_copy(ey_h, v_ey, sem),
             pltpu.async_copy(ez_h, v_ez, sem),
             pltpu.async_copy(sends.at[lsl], v_send, sem)]
    for src, dst in ((px_h, v_le0), (py_h, v_le1), (pz_h, v_le2),
                     (vx_h, v_le3), (vy_h, v_le4), (vz_h, v_le5)):
        descs.append(pltpu.async_copy(src.at[lsl], dst, sem))
    for j in range(NCH):
        descs.append(pltpu.async_copy(recvs.at[pl.ds(base + j * CH, CH)],
                                      v_idx.at[j], sem))

    for r in range(16):
        for k in range(BD // 16):
            v_zero[r, pl.ds(k * 16, 16)] = z16
    for t in range(ZR // 16):
        pltpu.sync_copy(v_zero, sh_acc.at[pl.ds(sid * ZR + t * 16, 16)])
    def _hzero(i, _):
        v_hist[pl.ds(i * 16, 16)] = z16i
        return 0

    lax.fori_loop(0, HP // 16, _hzero, 0)

    def _count(i, _):
        sl6 = pl.ds(i * 16, 16)
        k16 = v_recv6[sl6]
        e16 = base6 + i * 16 + iota
        ks, vs = plsc.sort_key_val(k16, e16)
        v_tmp16[...] = ks
        prev = plsc.load_gather(v_tmp16, [jnp.maximum(iota - 1, 0)])
        nxt = plsc.load_gather(v_tmp16, [jnp.minimum(iota + 1, 15)])
        isnew = (iota == 0) | (ks != prev)
        rk = iota - plsc.cummax(jnp.where(isnew, iota, 0))
        last = (iota == 15) | (ks != nxt)
        plsc.addupdate_scatter(v_hist, [ks], rk + 1, mask=last)
        v_key6[sl6] = ks
        v_val6[sl6] = vs
        v_rank6[sl6] = rk
        return 0

    lax.fori_loop(0, NE6 // 16, _count, 0)
    pltpu.sync_copy(v_hist, sh_hist.at[pl.ds(sid * HP, HP)])
    for d in descs:
        d.wait()
    plsc.subcore_barrier()

    ld = pltpu.async_copy(bf2.at[pl.ds(base, CH)], v_bf2.at[0], sem)
    for j in range(NCH):
        ld.wait()
        if j + 1 < NCH:
            ld = pltpu.async_copy(bf2.at[pl.ds(base + (j + 1) * CH, CH)],
                                  v_bf2.at[(j + 1) % 2], sem)
        pltpu.sync_copy(v_bf2.at[j % 2], sh_acc.at[v_idx.at[j]], add=True)

    carry = jnp.int32(0)
    for b in range(HP // HB):
        descs = [pltpu.async_copy(sh_hist.at[pl.ds(sp * HP + b * HB, HB)],
                                  v_allh.at[pl.ds(sp * HB, HB)], sem)
                 for sp in range(16)]
        for d in descs:
            d.wait()

        def _prefix(i, c):
            tot = z16i
            mysum = z16i
            for sp in range(16):
                h = v_allh[pl.ds(sp * HB + i * 16, 16)]
                tot = tot + h
                mysum = mysum + jnp.where(sp < sid, h, z16i)
            cum = plsc.cumsum(tot)
            excl = c + cum - tot
            v_off[pl.ds(b * HB + i * 16, 16)] = excl
            v_start[pl.ds(b * HB + i * 16, 16)] = excl + mysum
            return c + cum[15]

        carry = lax.fori_loop(0, HB // 16, _prefix, carry)

    def _place(i, _):
        sl6 = pl.ds(i * 16, 16)
        ks = v_key6[sl6]
        rk = v_rank6[sl6]
        st = plsc.load_gather(v_start, [ks])
        v_pos6[sl6] = st + rk
        v_tmp16[...] = ks
        nxt = plsc.load_gather(v_tmp16, [jnp.minimum(iota + 1, 15)])
        last = (iota == 15) | (ks != nxt)
        plsc.addupdate_scatter(v_start, [ks], rk + 1, mask=last)
        return 0

    lax.fori_loop(0, NE6 // 16, _place, 0)
    pltpu.sync_copy(v_pos6, sh_pos.at[pl.ds(base6, NE6)])
    pltpu.sync_copy(v_val6, sh_val.at[pl.ds(base6, NE6)])
    plsc.subcore_barrier()

    pltpu.sync_copy(sh_acc.at[pl.ds(sid * ZR, ZR)],
                    partials.at[cid, pl.ds(sid * ZR, ZR)])

    for j in range(EP // PCH):
        pltpu.sync_copy(sh_pos.at[pl.ds(j * PCH, PCH)], v_pch)
        pltpu.sync_copy(sh_val.at[pl.ds(j * PCH, PCH)], v_vch)

        def _scat(i, _):
            slc = pl.ds(i * 16, 16)
            plsc.store_scatter(v_perm, [v_pch[slc]], v_vch[slc])
            return 0

        lax.fori_loop(0, PCH // 16, _scat, 0)

    def _group(g, _):
        sl = pl.ds(g * 16, 16)
        n = v_send[sl]
        lo = plsc.load_gather(v_off, [n])
        hi = plsc.load_gather(v_off, [n + 1])
        px = v_le0[sl]
        py = v_le1[sl]
        pz = v_le2[sl]
        vx = v_le3[sl]
        vy = v_le4[sl]
        vz = v_le5[sl]
        ln = hi - lo
        maxlen = jnp.max(ln)

        def _cond(carry):
            return carry[0] < maxlen

        def _step(carry):
            t, accp, acct = carry
            j = lo + t
            msk = j < hi
            jc = jnp.where(msk, j, 0)
            e = plsc.load_gather(v_perm, [jc])
            ex = plsc.load_gather(v_ex, [e])
            ey = plsc.load_gather(v_ey, [e])
            ez = plsc.load_gather(v_ez, [e])
            cx = py * ez - pz * ey
            cy = pz * ex - px * ez
            cz = px * ey - py * ex
            c = _sqrt16(cx * cx + cy * cy + cz * cz)
            d1 = px * ex + py * ey + pz * ez
            d2 = vx * ex + vy * ey + vz * ez
            theta = _atan2_pos(c, d1)
            phi = _atan2_pos(jnp.abs(d1) * c, d1 * d2)
            zero = jnp.float32(0.0)
            accp = accp + jnp.where(msk, phi, zero)
            acct = acct + jnp.where(msk, theta, zero)
            return t + 1, accp, acct

        _, accp, acct = lax.while_loop(_cond, _step, (jnp.int32(0), z16, z16))
        v_p0[sl] = accp
        v_p1[sl] = acct
        v_p2[sl] = ln.astype(jnp.float32)
        return 0

    lax.fori_loop(0, EPW // 16, _group, 0)

    pltpu.sync_copy(v_p0, pout0.at[pl.ds(base, EPW)])
    pltpu.sync_copy(v_p1, pout1.at[pl.ds(base, EPW)])
    pltpu.sync_copy(v_p2, pout2.at[pl.ds(base, EPW)])


def _sc2(bf2, recvs_p, sends_p, le_cols, edge_cols):
    mesh = plsc.VectorSubcoreMesh(core_axis_name="c", subcore_axis_name="s")
    return pl.kernel(
        _sc2_body,
        out_type=[
            jax.ShapeDtypeStruct((2, NR, BD), jnp.float32),
            jax.ShapeDtypeStruct((EP,), jnp.float32),
            jax.ShapeDtypeStruct((EP,), jnp.float32),
            jax.ShapeDtypeStruct((EP,), jnp.float32),
        ],
        mesh=mesh,
        compiler_params=pltpu.CompilerParams(needs_layout_passes=False),
        scratch_types=[
            pltpu.VMEM_SHARED((NR, BD), jnp.float32),
            pltpu.VMEM_SHARED((16 * HP,), jnp.int32),
            pltpu.VMEM_SHARED((EP,), jnp.int32),
            pltpu.VMEM_SHARED((EP,), jnp.int32),
            pltpu.VMEM((2, CH, BD), jnp.float32),
            pltpu.VMEM((16, BD), jnp.float32),
            pltpu.VMEM((NCH, CH), jnp.int32),
            pltpu.VMEM((E,), jnp.float32),
            pltpu.VMEM((E,), jnp.float32),
            pltpu.VMEM((E,), jnp.float32),
            pltpu.VMEM((EP,), jnp.int32),
            pltpu.VMEM((HP,), jnp.int32),
            pltpu.VMEM((HP,), jnp.int32),
            pltpu.VMEM((HP,), jnp.int32),
            pltpu.VMEM((16 * HB,), jnp.int32),
            pltpu.VMEM((NE6,), jnp.int32),
            pltpu.VMEM((NE6,), jnp.int32),
            pltpu.VMEM((NE6,), jnp.int32),
            pltpu.VMEM((NE6,), jnp.int32),
            pltpu.VMEM((NE6,), jnp.int32),
            pltpu.VMEM((16,), jnp.int32),
            pltpu.VMEM((PCH,), jnp.int32),
            pltpu.VMEM((PCH,), jnp.int32),
            pltpu.VMEM((EPW,), jnp.float32),
            pltpu.VMEM((EPW,), jnp.float32),
            pltpu.VMEM((EPW,), jnp.float32),
            pltpu.VMEM((EPW,), jnp.float32),
            pltpu.VMEM((EPW,), jnp.float32),
            pltpu.VMEM((EPW,), jnp.float32),
            pltpu.VMEM((EPW,), jnp.int32),
            pltpu.VMEM((EPW,), jnp.float32),
            pltpu.VMEM((EPW,), jnp.float32),
            pltpu.VMEM((EPW,), jnp.float32),
            pltpu.SemaphoreType.DMA,
        ],
    )(bf2, recvs_p, sends_p, *le_cols, *edge_cols)


def _sc3_body(partials, sends, p0_h, p1_h, p2_h, aux, outp,
              v_r0, v_r1, v_idx, v_phi, v_th, v_cnt, v_aux, sem):
    cid = lax.axis_index("c")
    sid = lax.axis_index("s")
    wid = cid * 16 + sid
    base = wid * EPW

    descs = [pltpu.async_copy(sends.at[pl.ds(base + j * CH, CH)],
                              v_idx.at[j], sem) for j in range(NCH)]
    descs.append(pltpu.async_copy(p0_h.at[pl.ds(base, EPW)],
                                  v_phi.at[pl.ds(0, EPW)], sem))
    descs.append(pltpu.async_copy(p1_h.at[pl.ds(base, EPW)],
                                  v_th.at[pl.ds(0, EPW)], sem))
    descs.append(pltpu.async_copy(p2_h.at[pl.ds(base, EPW)],
                                  v_cnt.at[pl.ds(0, EPW)], sem))
    descs.append(pltpu.async_copy(aux, v_aux, sem))
    for d in descs:
        d.wait()
    gdescs = []
    for j in range(NCH):
        gdescs.append(pltpu.async_copy(partials.at[0].at[v_idx.at[j]],
                                       v_r0.at[pl.ds(j * CH, CH)], sem))
        gdescs.append(pltpu.async_copy(partials.at[1].at[v_idx.at[j]],
                                       v_r1.at[pl.ds(j * CH, CH)], sem))
    for d in gdescs:
        d.wait()

    a0 = [v_aux[0, pl.ds(k * 16, 16)] for k in range(BD // 16)]
    a1 = [v_aux[1, pl.ds(k * 16, 16)] for k in range(BD // 16)]
    a2 = [v_aux[2, pl.ds(k * 16, 16)] for k in range(BD // 16)]
    a3 = [v_aux[3, pl.ds(k * 16, 16)] for k in range(BD // 16)]

    def _row(r, _):
        sp = v_phi[pl.ds(r, 16)][0]
        st = v_th[pl.ds(r, 16)][0]
        sc = v_cnt[pl.ds(r, 16)][0]
        for k in range(BD // 16):
            sl = pl.ds(k * 16, 16)
            acc = v_r0[r, sl] + v_r1[r, sl]
            acc = acc + sp * a0[k] + st * a1[k] + sc * a2[k] + a3[k]
            v_r0[r, sl] = acc
        return 0

    lax.fori_loop(0, EPW, _row, 0)
    pltpu.sync_copy(v_r0, outp.at[pl.ds(base, EPW)])


def _sc3(partials, sends_p, p0, p1, p2, aux):
    mesh = plsc.VectorSubcoreMesh(core_axis_name="c", subcore_axis_name="s")
    return pl.kernel(
        _sc3_body,
        out_type=jax.ShapeDtypeStruct((EP, BD), jnp.float32),
        mesh=mesh,
        scratch_types=[
            pltpu.VMEM((EPW, BD), jnp.float32),
            pltpu.VMEM((EPW, BD), jnp.float32),
            pltpu.VMEM((NCH, CH), jnp.int32),
            pltpu.VMEM((EPW + 16,), jnp.float32),
            pltpu.VMEM((EPW + 16,), jnp.float32),
            pltpu.VMEM((EPW + 16,), jnp.float32),
            pltpu.VMEM((8, BD), jnp.float32),
            pltpu.SemaphoreType.DMA,
        ],
    )(partials, sends_p, p0, p1, p2, aux)


def kernel(local_env, pair_indices, bond_features, W_az, b_az, W_cat, b_cat):
    sends = pair_indices[:, 0].astype(jnp.int32)
    recvs = pair_indices[:, 1].astype(jnp.int32)

    pad_e = EP - E
    bond_p = jnp.pad(bond_features, ((0, pad_e), (0, 0)))
    sends_p = jnp.pad(sends, (0, pad_e))
    recvs_p = jnp.pad(recvs, (0, pad_e), constant_values=N_NODES)
    le_cols = tuple(jnp.pad(local_env[:, k], (0, pad_e)) for k in range(6))
    edge_cols = tuple(local_env[:, k] for k in range(3))

    wtop = W_cat[:BD]
    wbot = W_cat[BD:]
    m8 = jnp.zeros((8, 64), jnp.float32).at[0:2].set(W_az).at[2].set(b_az)
    bcat_row = b_cat.reshape(1, BD)

    bf2, aux = _tc1(bond_p, wtop, m8, wbot, bcat_row)
    partials, p0, p1, p2 = _sc2(bf2, recvs_p, sends_p, le_cols, edge_cols)
    outp = _sc3(partials, sends_p, p0, p1, p2, aux)
    return outp[:E]


# speedup vs baseline: 627.3768x; 1.3839x over previous
import functools

import jax
import jax.numpy as jnp
from jax import lax
from jax.experimental import pallas as pl
from jax.experimental.pallas import tpu as pltpu
from jax.experimental.pallas import tpu_sc as plsc

E = 10000
N_NODES = 2500
BD = 128
EP = 10240
NW = 32
EPW = EP // NW
CH = 64
NCH = EPW // CH
NR = 2560
ZR = NR // 16
OFFP = 2504

_ATAN_C = (0.9999994160035325, -0.3333022235532037, 0.19951110891900356,
           -0.13933229393279548, 0.09709350737147433, -0.05688089274197976,
           0.02256682612663299, -0.004257409078051173)
_PI = 3.14159265358979
_PI_2 = 1.570796326794897


def _sqrt16(x):
    i = plsc.bitcast(x, jnp.int32)
    y = plsc.bitcast(jnp.int32(0x1FBD1DF5) + lax.shift_right_arithmetic(i, 1),
                     jnp.float32)
    half = jnp.float32(0.5)
    for _ in range(3):
        y = half * (y + x / jnp.where(y == 0.0, jnp.float32(1.0), y))
    return jnp.where(x <= 0.0, jnp.float32(0.0), y)


def _atan2_pos(y, x):
    ax = jnp.abs(x)
    mx = jnp.maximum(ax, y)
    mn = jnp.minimum(ax, y)
    t = mn / jnp.where(mx == 0.0, jnp.float32(1.0), mx)
    u = t * t
    p = jnp.float32(_ATAN_C[7])
    for c in _ATAN_C[6::-1]:
        p = p * u + jnp.float32(c)
    r = t * p
    r = jnp.where(y > ax, jnp.float32(_PI_2) - r, r)
    r = jnp.where(x < 0.0, jnp.float32(_PI) - r, r)
    return jnp.where(mx == 0.0, jnp.float32(0.0), r)


NE6 = 640
HP = 2560
HB = 640
PCH = 2048


def _sc2_body(bond, recvs, sends, px_h, py_h, pz_h, vx_h, vy_h, vz_h,
              gout, p8out,
              sh_acc, sh_hist, sh_pos, sh_val,
              v_bf2, v_zero, v_idx, v_sidx, v_ex, v_ey, v_ez, v_perm,
              v_off, v_start, v_hist, v_allh, v_recv6, v_key6, v_val6,
              v_rank6, v_pos6, v_tmp16, v_pch, v_vch,
              v_le0, v_le1, v_le2, v_le3, v_le4, v_le5, v_send,
              v_p8, sem):
    cid = lax.axis_index("c")
    sid = lax.axis_index("s")
    wid = cid * 16 + sid
    base = wid * EPW
    base6 = sid * NE6
    iota = lax.broadcasted_iota(jnp.int32, (16,), 0)
    z16 = jnp.zeros((16,), jnp.float32)
    z16i = jnp.zeros((16,), jnp.int32)

    pltpu.sync_copy(recvs.at[pl.ds(base6, NE6)], v_recv6)

    lsl = pl.ds(base, EPW)
    esl = pl.ds(0, E)
    descs = [pltpu.async_copy(px_h.at[esl], v_ex, sem),
             pltpu.async_copy(py_h.at[esl], v_ey, sem),
             pltpu.async_copy(pz_h.at[esl], v_ez, sem),
             pltpu.async_copy(sends.at[lsl], v_send, sem)]
    for src, dst in ((px_h, v_le0), (py_h, v_le1), (pz_h, v_le2),
                     (vx_h, v_le3), (vy_h, v_le4), (vz_h, v_le5)):
        descs.append(pltpu.async_copy(src.at[lsl], dst, sem))
    for j in range(NCH):
        descs.append(pltpu.async_copy(recvs.at[pl.ds(base + j * CH, CH)],
                                      v_idx.at[j], sem))
    for j in range(NE6 // CH):
        descs.append(pltpu.async_copy(sends.at[pl.ds(base6 + j * CH, CH)],
                                      v_sidx.at[j], sem))

    for r in range(16):
        for k in range(BD // 16):
            v_zero[r, pl.ds(k * 16, 16)] = z16
    for t in range(ZR // 16):
        pltpu.sync_copy(v_zero, sh_acc.at[pl.ds(sid * ZR + t * 16, 16)])
    def _hzero(i, _):
        v_hist[pl.ds(i * 16, 16)] = z16i
        return 0

    lax.fori_loop(0, HP // 16, _hzero, 0)

    def _count(i, _):
        sl6 = pl.ds(i * 16, 16)
        k16 = v_recv6[sl6]
        e16 = base6 + i * 16 + iota
        ks, vs = plsc.sort_key_val(k16, e16)
        v_tmp16[...] = ks
        prev = plsc.load_gather(v_tmp16, [jnp.maximum(iota - 1, 0)])
        nxt = plsc.load_gather(v_tmp16, [jnp.minimum(iota + 1, 15)])
        isnew = (iota == 0) | (ks != prev)
        rk = iota - plsc.cummax(jnp.where(isnew, iota, 0))
        last = (iota == 15) | (ks != nxt)
        plsc.addupdate_scatter(v_hist, [ks], rk + 1, mask=last)
        v_key6[sl6] = ks
        v_val6[sl6] = vs
        v_rank6[sl6] = rk
        return 0

    lax.fori_loop(0, NE6 // 16, _count, 0)
    pltpu.sync_copy(v_hist, sh_hist.at[pl.ds(sid * HP, HP)])
    for d in descs:
        d.wait()
    plsc.subcore_barrier()

    ld = pltpu.async_copy(bond.at[pl.ds(base, CH)], v_bf2.at[0], sem)
    for j in range(NCH):
        ld.wait()
        if j + 1 < NCH:
            ld = pltpu.async_copy(bond.at[pl.ds(base + (j + 1) * CH, CH)],
                                  v_bf2.at[(j + 1) % 2], sem)
        pltpu.sync_copy(v_bf2.at[j % 2], sh_acc.at[v_idx.at[j]], add=True)

    carry = jnp.int32(0)
    for b in range(HP // HB):
        descs = [pltpu.async_copy(sh_hist.at[pl.ds(sp * HP + b * HB, HB)],
                                  v_allh.at[pl.ds(sp * HB, HB)], sem)
                 for sp in range(16)]
        for d in descs:
            d.wait()

        def _prefix(i, c):
            tot = z16i
            mysum = z16i
            for sp in range(16):
                h = v_allh[pl.ds(sp * HB + i * 16, 16)]
                tot = tot + h
                mysum = mysum + jnp.where(sp < sid, h, z16i)
            cum = plsc.cumsum(tot)
            excl = c + cum - tot
            v_off[pl.ds(b * HB + i * 16, 16)] = excl
            v_start[pl.ds(b * HB + i * 16, 16)] = excl + mysum
            return c + cum[15]

        carry = lax.fori_loop(0, HB // 16, _prefix, carry)

    def _place(i, _):
        sl6 = pl.ds(i * 16, 16)
        ks = v_key6[sl6]
        rk = v_rank6[sl6]
        st = plsc.load_gather(v_start, [ks])
        v_pos6[sl6] = st + rk
        v_tmp16[...] = ks
        nxt = plsc.load_gather(v_tmp16, [jnp.minimum(iota + 1, 15)])
        last = (iota == 15) | (ks != nxt)
        plsc.addupdate_scatter(v_start, [ks], rk + 1, mask=last)
        return 0

    lax.fori_loop(0, NE6 // 16, _place, 0)
    pltpu.sync_copy(v_pos6, sh_pos.at[pl.ds(base6, NE6)])
    pltpu.sync_copy(v_val6, sh_val.at[pl.ds(base6, NE6)])
    plsc.subcore_barrier()

    gd = pltpu.async_copy(sh_acc.at[v_sidx.at[0]], v_bf2.at[0], sem)
    for j in range(NE6 // CH):
        gd.wait()
        if j + 1 < NE6 // CH:
            gd = pltpu.async_copy(sh_acc.at[v_sidx.at[j + 1]],
                                  v_bf2.at[(j + 1) % 2], sem)
        pltpu.sync_copy(v_bf2.at[j % 2],
                        gout.at[cid, pl.ds(base6 + j * CH, CH)])

    for j in range(EP // PCH):
        pltpu.sync_copy(sh_pos.at[pl.ds(j * PCH, PCH)], v_pch)
        pltpu.sync_copy(sh_val.at[pl.ds(j * PCH, PCH)], v_vch)

        def _scat(i, _):
            slc = pl.ds(i * 16, 16)
            plsc.store_scatter(v_perm, [v_pch[slc]], v_vch[slc])
            return 0

        lax.fori_loop(0, PCH // 16, _scat, 0)

    def _p8zero(i, _):
        v_p8[pl.ds(i * 16, 16)] = z16
        return 0

    lax.fori_loop(0, EPW * 8 // 16, _p8zero, 0)

    def _group(g, _):
        sl = pl.ds(g * 16, 16)
        n = v_send[sl]
        lo = plsc.load_gather(v_off, [n])
        hi = plsc.load_gather(v_off, [n + 1])
        px = v_le0[sl]
        py = v_le1[sl]
        pz = v_le2[sl]
        vx = v_le3[sl]
        vy = v_le4[sl]
        vz = v_le5[sl]
        ln = hi - lo
        maxlen = jnp.max(ln)

        def _cond(carry):
            return carry[0] < maxlen

        def _step(carry):
            t, accp, acct = carry
            j = lo + t
            msk = j < hi
            jc = jnp.where(msk, j, 0)
            e = plsc.load_gather(v_perm, [jc])
            ex = plsc.load_gather(v_ex, [e])
            ey = plsc.load_gather(v_ey, [e])
            ez = plsc.load_gather(v_ez, [e])
            cx = py * ez - pz * ey
            cy = pz * ex - px * ez
            cz = px * ey - py * ex
            c = _sqrt16(cx * cx + cy * cy + cz * cz)
            d1 = px * ex + py * ey + pz * ez
            d2 = vx * ex + vy * ey + vz * ez
            theta = _atan2_pos(c, d1)
            phi = _atan2_pos(jnp.abs(d1) * c, d1 * d2)
            zero = jnp.float32(0.0)
            accp = accp + jnp.where(msk, phi, zero)
            acct = acct + jnp.where(msk, theta, zero)
            return t + 1, accp, acct

        _, accp, acct = lax.while_loop(_cond, _step, (jnp.int32(0), z16, z16))
        col = (g * 16 + iota) * 8
        plsc.store_scatter(v_p8, [col], accp)
        plsc.store_scatter(v_p8, [col + 1], acct)
        plsc.store_scatter(v_p8, [col + 2], ln.astype(jnp.float32))
        plsc.store_scatter(v_p8, [col + 3], jnp.full((16,), 1.0, jnp.float32))
        return 0

    lax.fori_loop(0, EPW // 16, _group, 0)

    pltpu.sync_copy(v_p8, p8out.at[pl.ds(base * 8, EPW * 8)])


def _sc2(bond_p, recvs_p, sends_p, le_cols):
    mesh = plsc.VectorSubcoreMesh(core_axis_name="c", subcore_axis_name="s")
    return pl.kernel(
        _sc2_body,
        out_type=[
            jax.ShapeDtypeStruct((2, EP, BD), jnp.float32),
            jax.ShapeDtypeStruct((EP * 8,), jnp.float32),
        ],
        mesh=mesh,
        compiler_params=pltpu.CompilerParams(needs_layout_passes=False),
        scratch_types=[
            pltpu.VMEM_SHARED((NR, BD), jnp.float32),
            pltpu.VMEM_SHARED((16 * HP,), jnp.int32),
            pltpu.VMEM_SHARED((EP,), jnp.int32),
            pltpu.VMEM_SHARED((EP,), jnp.int32),
            pltpu.VMEM((2, CH, BD), jnp.float32),
            pltpu.VMEM((16, BD), jnp.float32),
            pltpu.VMEM((NCH, CH), jnp.int32),
            pltpu.VMEM((NE6 // CH, CH), jnp.int32),
            pltpu.VMEM((E,), jnp.float32),
            pltpu.VMEM((E,), jnp.float32),
            pltpu.VMEM((E,), jnp.float32),
            pltpu.VMEM((EP,), jnp.int32),
            pltpu.VMEM((HP,), jnp.int32),
            pltpu.VMEM((HP,), jnp.int32),
            pltpu.VMEM((HP,), jnp.int32),
            pltpu.VMEM((16 * HB,), jnp.int32),
            pltpu.VMEM((NE6,), jnp.int32),
            pltpu.VMEM((NE6,), jnp.int32),
            pltpu.VMEM((NE6,), jnp.int32),
            pltpu.VMEM((NE6,), jnp.int32),
            pltpu.VMEM((NE6,), jnp.int32),
            pltpu.VMEM((16,), jnp.int32),
            pltpu.VMEM((PCH,), jnp.int32),
            pltpu.VMEM((PCH,), jnp.int32),
            pltpu.VMEM((EPW,), jnp.float32),
            pltpu.VMEM((EPW,), jnp.float32),
            pltpu.VMEM((EPW,), jnp.float32),
            pltpu.VMEM((EPW,), jnp.float32),
            pltpu.VMEM((EPW,), jnp.float32),
            pltpu.VMEM((EPW,), jnp.float32),
            pltpu.VMEM((EPW,), jnp.int32),
            pltpu.VMEM((EPW * 8,), jnp.float32),
            pltpu.SemaphoreType.DMA,
        ],
    )(bond_p, recvs_p, sends_p, *le_cols)


def _tcf_body(g_ref, p8_ref, wtop_ref, m8_ref, wbot_ref, bcat_ref, out_ref):
    mm = jnp.dot(m8_ref[...], wbot_ref[...], preferred_element_type=jnp.float32)
    rowid = lax.broadcasted_iota(jnp.int32, (8, 1), 0)
    aux = mm + jnp.where(rowid == 3, jnp.float32(1.0),
                         jnp.float32(0.0)) * bcat_ref[...]
    gsum = g_ref[0] + g_ref[1]
    out_ref[...] = (jnp.dot(gsum, wtop_ref[...],
                            preferred_element_type=jnp.float32)
                    + jnp.dot(p8_ref[...], aux,
                              preferred_element_type=jnp.float32))


def _tcf(g, p8, wtop, m8, wbot, bcat_row):
    blk = 512
    return pl.pallas_call(
        _tcf_body,
        grid=(EP // blk,),
        in_specs=[
            pl.BlockSpec((2, blk, BD), lambda i: (0, i, 0)),
            pl.BlockSpec((blk, 8), lambda i: (i, 0)),
            pl.BlockSpec((BD, BD), lambda i: (0, 0)),
            pl.BlockSpec((8, 64), lambda i: (0, 0)),
            pl.BlockSpec((64, BD), lambda i: (0, 0)),
            pl.BlockSpec((1, BD), lambda i: (0, 0)),
        ],
        out_specs=pl.BlockSpec((blk, BD), lambda i: (i, 0)),
        out_shape=jax.ShapeDtypeStruct((EP, BD), jnp.float32),
    )(g, p8, wtop, m8, wbot, bcat_row)


def kernel(local_env, pair_indices, bond_features, W_az, b_az, W_cat, b_cat):
    sends = pair_indices[:, 0].astype(jnp.int32)
    recvs = pair_indices[:, 1].astype(jnp.int32)

    pad_e = EP - E
    bond_p = jnp.pad(bond_features, ((0, pad_e), (0, 0)))
    sends_p = jnp.pad(sends, (0, pad_e))
    recvs_p = jnp.pad(recvs, (0, pad_e), constant_values=N_NODES)
    le_cols = tuple(jnp.pad(local_env[:, k], (0, pad_e)) for k in range(6))

    wtop = W_cat[:BD]
    wbot = W_cat[BD:]
    m8 = jnp.zeros((8, 64), jnp.float32).at[0:2].set(W_az).at[2].set(b_az)
    bcat_row = b_cat.reshape(1, BD)

    g, p8 = _sc2(bond_p, recvs_p, sends_p, le_cols)
    outp = _tcf(g, p8.reshape(EP, 8), wtop, m8, wbot, bcat_row)
    return outp[:E]


# speedup vs baseline: 627.6598x; 1.0005x over previous
import jax
import jax.numpy as jnp
from jax import lax
from jax.experimental import pallas as pl
from jax.experimental.pallas import tpu as pltpu
from jax.experimental.pallas import tpu_sc as plsc

E = 10000
N_NODES = 2500
BD = 128
EP = 10240
NW = 32
EPW = EP // NW
CH = 64
NCH = EPW // CH
NR = 2560
ZR = NR // 16

_ATAN_C = (0.9999994160035325, -0.3333022235532037, 0.19951110891900356,
           -0.13933229393279548, 0.09709350737147433, -0.05688089274197976,
           0.02256682612663299, -0.004257409078051173)
_PI = 3.14159265358979
_PI_2 = 1.570796326794897


def _sqrt16(x):
    i = plsc.bitcast(x, jnp.int32)
    y = plsc.bitcast(jnp.int32(0x1FBD1DF5) + lax.shift_right_arithmetic(i, 1),
                     jnp.float32)
    half = jnp.float32(0.5)
    for _ in range(3):
        y = half * (y + x / jnp.where(y == 0.0, jnp.float32(1.0), y))
    return jnp.where(x <= 0.0, jnp.float32(0.0), y)


def _atan2_pos(y, x):
    ax = jnp.abs(x)
    mx = jnp.maximum(ax, y)
    mn = jnp.minimum(ax, y)
    t = mn / jnp.where(mx == 0.0, jnp.float32(1.0), mx)
    u = t * t
    p = jnp.float32(_ATAN_C[7])
    for c in _ATAN_C[6::-1]:
        p = p * u + jnp.float32(c)
    r = t * p
    r = jnp.where(y > ax, jnp.float32(_PI_2) - r, r)
    r = jnp.where(x < 0.0, jnp.float32(_PI) - r, r)
    return jnp.where(mx == 0.0, jnp.float32(0.0), r)


NE6 = 640
HP = 2560
HB = 640
PCH = 2048


def _sc2_body(bond, recvs, sends, px_h, py_h, pz_h, vx_h, vy_h, vz_h,
              gout, p8out,
              sh_acc, sh_hist, sh_pos, sh_val,
              v_bf2, v_zero, v_idx, v_sidx, v_ex, v_ey, v_ez, v_perm,
              v_off, v_start, v_hist, v_allh, v_recv6, v_key6, v_val6,
              v_rank6, v_pos6, v_tmp16, v_pch, v_vch,
              v_le0, v_le1, v_le2, v_le3, v_le4, v_le5, v_send,
              v_p8, sem):
    cid = lax.axis_index("c")
    sid = lax.axis_index("s")
    wid = cid * 16 + sid
    base = wid * EPW
    base6 = sid * NE6
    iota = lax.broadcasted_iota(jnp.int32, (16,), 0)
    z16 = jnp.zeros((16,), jnp.float32)
    z16i = jnp.zeros((16,), jnp.int32)

    pltpu.sync_copy(recvs.at[pl.ds(base6, NE6)], v_recv6)

    lsl = pl.ds(base, EPW)
    esl = pl.ds(0, E)
    descs = [pltpu.async_copy(px_h.at[esl], v_ex, sem),
             pltpu.async_copy(py_h.at[esl], v_ey, sem),
             pltpu.async_copy(pz_h.at[esl], v_ez, sem),
             pltpu.async_copy(sends.at[lsl], v_send, sem)]
    for src, dst in ((px_h, v_le0), (py_h, v_le1), (pz_h, v_le2),
                     (vx_h, v_le3), (vy_h, v_le4), (vz_h, v_le5)):
        descs.append(pltpu.async_copy(src.at[lsl], dst, sem))
    for j in range(NCH):
        descs.append(pltpu.async_copy(recvs.at[pl.ds(base + j * CH, CH)],
                                      v_idx.at[j], sem))
    for j in range(NE6 // CH):
        descs.append(pltpu.async_copy(sends.at[pl.ds(base6 + j * CH, CH)],
                                      v_sidx.at[j], sem))

    for r in range(16):
        for k in range(BD // 16):
            v_zero[r, pl.ds(k * 16, 16)] = z16
    for t in range(ZR // 16):
        pltpu.sync_copy(v_zero, sh_acc.at[pl.ds(sid * ZR + t * 16, 16)])
    def _hzero(i, _):
        v_hist[pl.ds(i * 16, 16)] = z16i
        return 0

    lax.fori_loop(0, HP // 16, _hzero, 0)

    def _count(i, _):
        sl6 = pl.ds(i * 16, 16)
        k16 = v_recv6[sl6]
        e16 = base6 + i * 16 + iota
        ks, vs = plsc.sort_key_val(k16, e16)
        v_tmp16[...] = ks
        prev = plsc.load_gather(v_tmp16, [jnp.maximum(iota - 1, 0)])
        nxt = plsc.load_gather(v_tmp16, [jnp.minimum(iota + 1, 15)])
        isnew = (iota == 0) | (ks != prev)
        rk = iota - plsc.cummax(jnp.where(isnew, iota, 0))
        last = (iota == 15) | (ks != nxt)
        plsc.addupdate_scatter(v_hist, [ks], rk + 1, mask=last)
        v_key6[sl6] = ks
        v_val6[sl6] = vs
        v_rank6[sl6] = rk
        return 0

    lax.fori_loop(0, NE6 // 16, _count, 0)
    pltpu.sync_copy(v_hist, sh_hist.at[pl.ds(sid * HP, HP)])
    for d in descs:
        d.wait()
    plsc.subcore_barrier()

    ld = pltpu.async_copy(bond.at[pl.ds(base, CH)], v_bf2.at[0], sem)
    for j in range(NCH):
        ld.wait()
        if j + 1 < NCH:
            ld = pltpu.async_copy(bond.at[pl.ds(base + (j + 1) * CH, CH)],
                                  v_bf2.at[(j + 1) % 2], sem)
        pltpu.sync_copy(v_bf2.at[j % 2], sh_acc.at[v_idx.at[j]], add=True)

    carry = jnp.int32(0)
    for b in range(HP // HB):
        descs = [pltpu.async_copy(sh_hist.at[pl.ds(sp * HP + b * HB, HB)],
                                  v_allh.at[pl.ds(sp * HB, HB)], sem)
                 for sp in range(16)]
        for d in descs:
            d.wait()

        def _prefix(i, c):
            tot = z16i
            mysum = z16i
            for sp in range(16):
                h = v_allh[pl.ds(sp * HB + i * 16, 16)]
                tot = tot + h
                mysum = mysum + jnp.where(sp < sid, h, z16i)
            cum = plsc.cumsum(tot)
            excl = c + cum - tot
            v_off[pl.ds(b * HB + i * 16, 16)] = excl
            v_start[pl.ds(b * HB + i * 16, 16)] = excl + mysum
            return c + cum[15]

        carry = lax.fori_loop(0, HB // 16, _prefix, carry)

    def _place(i, _):
        sl6 = pl.ds(i * 16, 16)
        ks = v_key6[sl6]
        rk = v_rank6[sl6]
        st = plsc.load_gather(v_start, [ks])
        v_pos6[sl6] = st + rk
        v_tmp16[...] = ks
        nxt = plsc.load_gather(v_tmp16, [jnp.minimum(iota + 1, 15)])
        last = (iota == 15) | (ks != nxt)
        plsc.addupdate_scatter(v_start, [ks], rk + 1, mask=last)
        return 0

    lax.fori_loop(0, NE6 // 16, _place, 0)
    pltpu.sync_copy(v_pos6, sh_pos.at[pl.ds(base6, NE6)])
    pltpu.sync_copy(v_val6, sh_val.at[pl.ds(base6, NE6)])
    plsc.subcore_barrier()

    gd = pltpu.async_copy(sh_acc.at[v_sidx.at[0]], v_bf2.at[0], sem)
    for j in range(NE6 // CH):
        gd.wait()
        if j + 1 < NE6 // CH:
            gd = pltpu.async_copy(sh_acc.at[v_sidx.at[j + 1]],
                                  v_bf2.at[(j + 1) % 2], sem)
        pltpu.sync_copy(v_bf2.at[j % 2],
                        gout.at[cid, pl.ds(base6 + j * CH, CH)])

    for j in range(EP // PCH):
        pltpu.sync_copy(sh_pos.at[pl.ds(j * PCH, PCH)], v_pch)
        pltpu.sync_copy(sh_val.at[pl.ds(j * PCH, PCH)], v_vch)

        def _scat(i, _):
            slc = pl.ds(i * 16, 16)
            plsc.store_scatter(v_perm, [v_pch[slc]], v_vch[slc])
            return 0

        lax.fori_loop(0, PCH // 16, _scat, 0)

    def _p8zero(i, _):
        v_p8[pl.ds(i * 16, 16)] = z16
        return 0

    lax.fori_loop(0, EPW * 8 // 16, _p8zero, 0)

    def _group(g, _):
        sl = pl.ds(g * 16, 16)
        n = v_send[sl]
        lo = plsc.load_gather(v_off, [n])
        hi = plsc.load_gather(v_off, [n + 1])
        px = v_le0[sl]
        py = v_le1[sl]
        pz = v_le2[sl]
        vx = v_le3[sl]
        vy = v_le4[sl]
        vz = v_le5[sl]
        ln = hi - lo
        maxlen = jnp.max(ln)

        def _cond(carry):
            return carry[0] < maxlen

        def _step(carry):
            t, accp, acct = carry
            j = lo + t
            msk = j < hi
            jc = jnp.where(msk, j, 0)
            e = plsc.load_gather(v_perm, [jc])
            ex = plsc.load_gather(v_ex, [e])
            ey = plsc.load_gather(v_ey, [e])
            ez = plsc.load_gather(v_ez, [e])
            cx = py * ez - pz * ey
            cy = pz * ex - px * ez
            cz = px * ey - py * ex
            c = _sqrt16(cx * cx + cy * cy + cz * cz)
            d1 = px * ex + py * ey + pz * ez
            d2 = vx * ex + vy * ey + vz * ez
            theta = _atan2_pos(c, d1)
            phi = _atan2_pos(jnp.abs(d1) * c, d1 * d2)
            zero = jnp.float32(0.0)
            accp = accp + jnp.where(msk, phi, zero)
            acct = acct + jnp.where(msk, theta, zero)
            return t + 1, accp, acct

        _, accp, acct = lax.while_loop(_cond, _step, (jnp.int32(0), z16, z16))
        col = (g * 16 + iota) * 8
        plsc.store_scatter(v_p8, [col], accp)
        plsc.store_scatter(v_p8, [col + 1], acct)
        plsc.store_scatter(v_p8, [col + 2], ln.astype(jnp.float32))
        plsc.store_scatter(v_p8, [col + 3], jnp.full((16,), 1.0, jnp.float32))
        return 0

    lax.fori_loop(0, EPW // 16, _group, 0)

    pltpu.sync_copy(v_p8, p8out.at[pl.ds(base * 8, EPW * 8)])


def _sc2(bond_p, recvs_p, sends_p, le_cols):
    mesh = plsc.VectorSubcoreMesh(core_axis_name="c", subcore_axis_name="s")
    return pl.kernel(
        _sc2_body,
        out_type=[
            jax.ShapeDtypeStruct((2, EP, BD), jnp.float32),
            jax.ShapeDtypeStruct((EP * 8,), jnp.float32),
        ],
        mesh=mesh,
        compiler_params=pltpu.CompilerParams(needs_layout_passes=False),
        scratch_types=[
            pltpu.VMEM_SHARED((NR, BD), jnp.float32),
            pltpu.VMEM_SHARED((16 * HP,), jnp.int32),
            pltpu.VMEM_SHARED((EP,), jnp.int32),
            pltpu.VMEM_SHARED((EP,), jnp.int32),
            pltpu.VMEM((2, CH, BD), jnp.float32),
            pltpu.VMEM((16, BD), jnp.float32),
            pltpu.VMEM((NCH, CH), jnp.int32),
            pltpu.VMEM((NE6 // CH, CH), jnp.int32),
            pltpu.VMEM((E,), jnp.float32),
            pltpu.VMEM((E,), jnp.float32),
            pltpu.VMEM((E,), jnp.float32),
            pltpu.VMEM((EP,), jnp.int32),
            pltpu.VMEM((HP,), jnp.int32),
            pltpu.VMEM((HP,), jnp.int32),
            pltpu.VMEM((HP,), jnp.int32),
            pltpu.VMEM((16 * HB,), jnp.int32),
            pltpu.VMEM((NE6,), jnp.int32),
            pltpu.VMEM((NE6,), jnp.int32),
            pltpu.VMEM((NE6,), jnp.int32),
            pltpu.VMEM((NE6,), jnp.int32),
            pltpu.VMEM((NE6,), jnp.int32),
            pltpu.VMEM((16,), jnp.int32),
            pltpu.VMEM((PCH,), jnp.int32),
            pltpu.VMEM((PCH,), jnp.int32),
            pltpu.VMEM((EPW,), jnp.float32),
            pltpu.VMEM((EPW,), jnp.float32),
            pltpu.VMEM((EPW,), jnp.float32),
            pltpu.VMEM((EPW,), jnp.float32),
            pltpu.VMEM((EPW,), jnp.float32),
            pltpu.VMEM((EPW,), jnp.float32),
            pltpu.VMEM((EPW,), jnp.int32),
            pltpu.VMEM((EPW * 8,), jnp.float32),
            pltpu.SemaphoreType.DMA,
        ],
    )(bond_p, recvs_p, sends_p, *le_cols)


def _tcf_body(g_ref, p8_ref, wtop_ref, m8_ref, wbot_ref, bcat_ref, out_ref):
    mm = jnp.dot(m8_ref[...], wbot_ref[...], preferred_element_type=jnp.float32)
    rowid = lax.broadcasted_iota(jnp.int32, (8, 1), 0)
    aux = mm + jnp.where(rowid == 3, jnp.float32(1.0),
                         jnp.float32(0.0)) * bcat_ref[...]
    gsum = g_ref[0] + g_ref[1]
    out_ref[...] = (jnp.dot(gsum, wtop_ref[...],
                            preferred_element_type=jnp.float32)
                    + jnp.dot(p8_ref[...], aux,
                              preferred_element_type=jnp.float32))


def _tcf(g, p8, wtop, m8, wbot, bcat_row):
    blk = 512
    return pl.pallas_call(
        _tcf_body,
        grid=(EP // blk,),
        in_specs=[
            pl.BlockSpec((2, blk, BD), lambda i: (0, i, 0)),
            pl.BlockSpec((blk, 8), lambda i: (i, 0)),
            pl.BlockSpec((BD, BD), lambda i: (0, 0)),
            pl.BlockSpec((8, 64), lambda i: (0, 0)),
            pl.BlockSpec((64, BD), lambda i: (0, 0)),
            pl.BlockSpec((1, BD), lambda i: (0, 0)),
        ],
        out_specs=pl.BlockSpec((blk, BD), lambda i: (i, 0)),
        out_shape=jax.ShapeDtypeStruct((EP, BD), jnp.float32),
    )(g, p8, wtop, m8, wbot, bcat_row)


def kernel(local_env, pair_indices, bond_features, W_az, b_az, W_cat, b_cat):
    sends = pair_indices[:, 0].astype(jnp.int32)
    recvs = pair_indices[:, 1].astype(jnp.int32)

    pad_e = EP - E
    bond_p = jnp.pad(bond_features, ((0, pad_e), (0, 0)))
    sends_p = jnp.pad(sends, (0, pad_e))
    recvs_p = jnp.pad(recvs, (0, pad_e), constant_values=N_NODES)
    le_cols = tuple(jnp.pad(local_env[:, k], (0, pad_e)) for k in range(6))

    wtop = W_cat[:BD]
    wbot = W_cat[BD:]
    m8 = jnp.zeros((8, 64), jnp.float32).at[0:2].set(W_az).at[2].set(b_az)
    bcat_row = b_cat.reshape(1, BD)

    g, p8 = _sc2(bond_p, recvs_p, sends_p, le_cols)
    outp = _tcf(g, p8.reshape(EP, 8), wtop, m8, wbot, bcat_row)
    return outp[:E]
